# Initial kernel scaffold; baseline (speedup 1.0000x reference)
import jax
import jax.numpy as jnp
from jax.experimental import pallas as pl


def kernel(x, W_rp, b_rp, W_rf, b_rf, W_emb, b_emb, cls_tok, ln1_g, ln1_b, Wqkv, bqkv, Wo, bo, ln2_g, ln2_b, W1, b1, W2, b2, lnf_g, lnf_b, Wh1, bh1, Wh2, bh2):
    raise NotImplementedError("write your pallas kernel here")


# speedup vs baseline: 1.8425x; 1.8425x over previous
import functools

import jax
import jax.numpy as jnp
from jax.experimental import pallas as pl
from jax.experimental.pallas import tpu as pltpu

E = 8
L = 2
D_IN = 1024
DIM = 512
HEADS = 8
DH = 64
MLP = 512
NC = 2
N = 1024
TP = 1032


def _ln(x, g, b, eps=1e-5):
    m = jnp.mean(x, axis=-1, keepdims=True)
    v = jnp.var(x, axis=-1, keepdims=True)
    return (x - m) / jnp.sqrt(v + eps) * g + b


def _dot(a, b):
    return jax.lax.dot(a.astype(jnp.bfloat16), b.astype(jnp.bfloat16),
                       preferred_element_type=jnp.float32)


def _dot_t(a, b):
    return jax.lax.dot_general(
        a.astype(jnp.bfloat16), b.astype(jnp.bfloat16),
        (((1,), (1,)), ((), ())), preferred_element_type=jnp.float32)


def _embed_kernel(x_ref, wrp_ref, brp_ref, wrf_ref, brf_ref,
                  wemb_ref, bemb_ref, hemb_out, g_out):
    e = pl.program_id(0)

    @pl.when(e == 0)
    def _router():
        r = jnp.maximum(_dot(x_ref[...], wrp_ref[...]) + brp_ref[...], 0.0)
        rm = jnp.mean(r, axis=0, keepdims=True)
        logits = _dot(rm, wrf_ref[...]) + brf_ref[...]
        g_out[...] = jax.nn.softmax(logits, axis=-1)

    hemb_out[...] = (_dot(x_ref[...], wemb_ref[0]) + bemb_ref[0])[None]


def _qkv_kernel(a_ref, wqkv_ref, bqkv_ref, q_out, k_out, v_out):
    qkv = _dot(a_ref[0], wqkv_ref[0]) + bqkv_ref[0]
    for hd in range(HEADS):
        q_out[0, hd] = qkv[:, hd * DH:(hd + 1) * DH]
        k_out[0, hd] = qkv[:, DIM + hd * DH:DIM + (hd + 1) * DH]
        v_out[0, hd] = qkv[:, 2 * DIM + hd * DH:2 * DIM + (hd + 1) * DH]


def _scores_kernel(q_ref, k_ref, s_out):
    kmask = jnp.where(
        jax.lax.broadcasted_iota(jnp.int32, (1, TP), 1) <= N, 0.0, -1e30)
    s = _dot_t(q_ref[0, 0], k_ref[0, 0]) / jnp.sqrt(jnp.float32(DH)) + kmask
    s_out[...] = s[None, None]


def _attv_kernel(p_ref, v_ref, h_ref, wo_ref, bo_ref, h_out, o_ref):
    hd = pl.program_id(1)
    for h0 in range(HEADS):
        @pl.when(hd == h0)
        def _store():
            o_ref[:, h0 * DH:(h0 + 1) * DH] = _dot(p_ref[0, 0], v_ref[0, 0])

    @pl.when(hd == HEADS - 1)
    def _proj():
        h_out[...] = (h_ref[0] + _dot(o_ref[...], wo_ref[0]) + bo_ref[0])[None]


def _qkv_call(a3, wqkv, bqkv):
    return pl.pallas_call(
        _qkv_kernel,
        grid=(E,),
        in_specs=[
            _expert_spec((TP, DIM)),
            _expert_spec((DIM, 3 * DIM)),
            _expert_spec((1, 3 * DIM)),
        ],
        out_specs=(_expert_spec((HEADS, TP, DH)),) * 3,
        out_shape=(jax.ShapeDtypeStruct((E, HEADS, TP, DH), jnp.float32),) * 3,
        compiler_params=pltpu.CompilerParams(
            dimension_semantics=("arbitrary",)),
    )(a3, wqkv, bqkv.reshape(E, 1, 3 * DIM))


def _scores_call(q, k):
    return pl.pallas_call(
        _scores_kernel,
        grid=(E, HEADS),
        in_specs=[
            pl.BlockSpec((1, 1, TP, DH), lambda e, hd: (e, hd, 0, 0)),
            pl.BlockSpec((1, 1, TP, DH), lambda e, hd: (e, hd, 0, 0)),
        ],
        out_specs=pl.BlockSpec((1, 1, TP, TP), lambda e, hd: (e, hd, 0, 0)),
        out_shape=jax.ShapeDtypeStruct((E, HEADS, TP, TP), jnp.float32),
        compiler_params=pltpu.CompilerParams(
            dimension_semantics=("arbitrary", "arbitrary")),
    )(q, k)


def _attv_call(p, v, h3, wo, bo):
    return pl.pallas_call(
        _attv_kernel,
        grid=(E, HEADS),
        in_specs=[
            pl.BlockSpec((1, 1, TP, TP), lambda e, hd: (e, hd, 0, 0)),
            pl.BlockSpec((1, 1, TP, DH), lambda e, hd: (e, hd, 0, 0)),
            pl.BlockSpec((1, TP, DIM), lambda e, hd: (e, 0, 0)),
            pl.BlockSpec((1, DIM, DIM), lambda e, hd: (e, 0, 0)),
            pl.BlockSpec((1, 1, DIM), lambda e, hd: (e, 0, 0)),
        ],
        out_specs=pl.BlockSpec((1, TP, DIM), lambda e, hd: (e, 0, 0)),
        out_shape=jax.ShapeDtypeStruct((E, TP, DIM), jnp.float32),
        scratch_shapes=[pltpu.VMEM((TP, DIM), jnp.float32)],
        compiler_params=pltpu.CompilerParams(
            dimension_semantics=("arbitrary", "arbitrary")),
    )(p, v, h3, wo, bo.reshape(E, 1, DIM))


def _attn_kernel(a_ref, h_ref, wqkv_ref, bqkv_ref, wo_ref, bo_ref, h_out):
    a = a_ref[0]
    qkv = _dot(a, wqkv_ref[0]) + bqkv_ref[0]
    kmask = jnp.where(
        jax.lax.broadcasted_iota(jnp.int32, (1, TP), 1) <= N, 0.0, -1e30)
    o_heads = []
    for hd in range(HEADS):
        q = qkv[:, hd * DH:(hd + 1) * DH]
        k = qkv[:, DIM + hd * DH:DIM + (hd + 1) * DH]
        v = qkv[:, 2 * DIM + hd * DH:2 * DIM + (hd + 1) * DH]
        s = _dot_t(q, k) / jnp.sqrt(jnp.float32(DH)) + kmask
        p = jax.nn.softmax(s, axis=-1)
        o_heads.append(_dot(p, v))
    o = jnp.concatenate(o_heads, axis=-1)
    h_out[...] = (h_ref[0] + (_dot(o, wo_ref[0]) + bo_ref[0]))[None]


def _mlp1_kernel(m_ref, w1_ref, b1_ref, u_out):
    u_out[...] = (_dot(m_ref[0], w1_ref[0]) + b1_ref[0])[None]


def _mlp2_kernel(g_ref, h_ref, w2_ref, b2_ref, h_out):
    h_out[...] = (h_ref[0] + (_dot(g_ref[0], w2_ref[0]) + b2_ref[0]))[None]


def _expert_spec(shape):
    n = len(shape)
    return pl.BlockSpec((1,) + shape, lambda e: (e,) + (0,) * n)


def _const_spec(shape):
    n = len(shape)
    return pl.BlockSpec(shape, lambda e: (0,) * n)


def _attn_call(a3, h3, wqkv, bqkv, wo, bo):
    return pl.pallas_call(
        _attn_kernel,
        grid=(E,),
        in_specs=[
            _expert_spec((TP, DIM)),
            _expert_spec((TP, DIM)),
            _expert_spec((DIM, 3 * DIM)),
            _expert_spec((1, 3 * DIM)),
            _expert_spec((DIM, DIM)),
            _expert_spec((1, DIM)),
        ],
        out_specs=_expert_spec((TP, DIM)),
        out_shape=jax.ShapeDtypeStruct((E, TP, DIM), jnp.float32),
        compiler_params=pltpu.CompilerParams(
            dimension_semantics=("arbitrary",)),
    )(a3, h3, wqkv, bqkv.reshape(E, 1, 3 * DIM), wo, bo.reshape(E, 1, DIM))


def _mlp_call(m3, h3, w1, b1, w2, b2):
    u = pl.pallas_call(
        _mlp1_kernel,
        grid=(E,),
        in_specs=[
            _expert_spec((TP, DIM)),
            _expert_spec((DIM, MLP)),
            _expert_spec((1, MLP)),
        ],
        out_specs=_expert_spec((TP, MLP)),
        out_shape=jax.ShapeDtypeStruct((E, TP, MLP), jnp.float32),
        compiler_params=pltpu.CompilerParams(
            dimension_semantics=("arbitrary",)),
    )(m3, w1, b1.reshape(E, 1, MLP))
    g = jax.nn.gelu(u)
    return pl.pallas_call(
        _mlp2_kernel,
        grid=(E,),
        in_specs=[
            _expert_spec((TP, MLP)),
            _expert_spec((TP, DIM)),
            _expert_spec((MLP, DIM)),
            _expert_spec((1, DIM)),
        ],
        out_specs=_expert_spec((TP, DIM)),
        out_shape=jax.ShapeDtypeStruct((E, TP, DIM), jnp.float32),
        compiler_params=pltpu.CompilerParams(
            dimension_semantics=("arbitrary",)),
    )(g, h3, w2, b2.reshape(E, 1, DIM))


@jax.jit
def _run(x, W_rp, b_rp, W_rf, b_rf, W_emb, b_emb, cls_tok, ln1_g, ln1_b,
         Wqkv, bqkv, Wo, bo, ln2_g, ln2_b, W1, b1, W2, b2, lnf_g, lnf_b,
         Wh1, bh1, Wh2, bh2):
    x2 = x.reshape(N, D_IN)

    hemb, g_soft = pl.pallas_call(
        _embed_kernel,
        grid=(E,),
        in_specs=[
            _const_spec((N, D_IN)),
            _const_spec((D_IN, DIM)),
            _const_spec((1, DIM)),
            _const_spec((DIM, E)),
            _const_spec((1, E)),
            _expert_spec((D_IN, DIM)),
            _expert_spec((1, DIM)),
        ],
        out_specs=(_expert_spec((N, DIM)), _const_spec((1, E))),
        out_shape=(jax.ShapeDtypeStruct((E, N, DIM), jnp.float32),
                   jax.ShapeDtypeStruct((1, E), jnp.float32)),
        compiler_params=pltpu.CompilerParams(
            dimension_semantics=("arbitrary",)),
    )(x2, W_rp, b_rp.reshape(1, DIM), W_rf, b_rf.reshape(1, E),
      W_emb, b_emb.reshape(E, 1, DIM))

    h = jnp.concatenate(
        [cls_tok[:, None, :], hemb,
         jnp.zeros((E, TP - N - 1, DIM), jnp.float32)], axis=1)
    h = jax.lax.optimization_barrier(h)

    for l in range(L):
        a = _ln(h, ln1_g[:, l][:, None, :], ln1_b[:, l][:, None, :])
        h = _attn_call(a, h, Wqkv[:, l], bqkv[:, l], Wo[:, l], bo[:, l])
        m = _ln(h, ln2_g[:, l][:, None, :], ln2_b[:, l][:, None, :])
        h = _mlp_call(m, h, W1[:, l], b1[:, l], W2[:, l], b2[:, l])

    hf = _ln(h, lnf_g[:, None, :], lnf_b[:, None, :])
    latents = hf[:, 0][None]
    latent = jnp.sum(latents * g_soft[..., None], axis=1)
    logits = jax.nn.relu(latent @ Wh1 + bh1) @ Wh2 + bh2
    return latent, logits, g_soft


def kernel(x, W_rp, b_rp, W_rf, b_rf, W_emb, b_emb, cls_tok, ln1_g, ln1_b,
           Wqkv, bqkv, Wo, bo, ln2_g, ln2_b, W1, b1, W2, b2, lnf_g, lnf_b,
           Wh1, bh1, Wh2, bh2):
    return _run(x, W_rp, b_rp, W_rf, b_rf, W_emb, b_emb, cls_tok, ln1_g,
                ln1_b, Wqkv, bqkv, Wo, bo, ln2_g, ln2_b, W1, b1, W2, b2,
                lnf_g, lnf_b, Wh1, bh1, Wh2, bh2)


# speedup vs baseline: 1.9732x; 1.0709x over previous
import functools

import jax
import jax.numpy as jnp
from jax.experimental import pallas as pl
from jax.experimental.pallas import tpu as pltpu

E = 8
L = 2
D_IN = 1024
DIM = 512
HEADS = 8
DH = 64
MLP = 512
NC = 2
N = 1024
TP = 1032


def _ln(x, g, b, eps=1e-5):
    m = jnp.mean(x, axis=-1, keepdims=True)
    v = jnp.var(x, axis=-1, keepdims=True)
    return (x - m) / jnp.sqrt(v + eps) * g + b


def _dot(a, b):
    return jax.lax.dot(a.astype(jnp.bfloat16), b.astype(jnp.bfloat16),
                       preferred_element_type=jnp.float32)


def _dot_t(a, b):
    return jax.lax.dot_general(
        a.astype(jnp.bfloat16), b.astype(jnp.bfloat16),
        (((1,), (1,)), ((), ())), preferred_element_type=jnp.float32)


def _embed_kernel(x_ref, wrp_ref, brp_ref, wrf_ref, brf_ref,
                  wemb_ref, bemb_ref, hemb_out, g_out):
    e = pl.program_id(0)

    @pl.when(e == 0)
    def _router():
        r = jnp.maximum(_dot(x_ref[...], wrp_ref[...]) + brp_ref[...], 0.0)
        rm = jnp.mean(r, axis=0, keepdims=True)
        logits = _dot(rm, wrf_ref[...]) + brf_ref[...]
        g_out[...] = jax.nn.softmax(logits, axis=-1)

    hemb_out[...] = (_dot(x_ref[...], wemb_ref[0]) + bemb_ref[0])[None]


def _attn_kernel(a_ref, h_ref, wqkv_ref, bqkv_ref, wo_ref, bo_ref, h_out):
    a = a_ref[0]
    qkv = _dot(a, wqkv_ref[0]) + bqkv_ref[0]
    kmask = jnp.where(
        jax.lax.broadcasted_iota(jnp.int32, (1, TP), 1) <= N, 0.0, -1e30)
    o_heads = []
    for hd in range(HEADS):
        q = qkv[:, hd * DH:(hd + 1) * DH]
        k = qkv[:, DIM + hd * DH:DIM + (hd + 1) * DH]
        v = qkv[:, 2 * DIM + hd * DH:2 * DIM + (hd + 1) * DH]
        s = _dot_t(q, k) / jnp.sqrt(jnp.float32(DH)) + kmask
        p = jax.nn.softmax(s, axis=-1)
        o_heads.append(_dot(p, v))
    o = jnp.concatenate(o_heads, axis=-1)
    h_out[...] = (h_ref[0] + (_dot(o, wo_ref[0]) + bo_ref[0]))[None]


def _mlp_kernel(m_ref, h_ref, w1_ref, b1_ref, w2_ref, b2_ref, h_out):
    u = jax.nn.gelu(_dot(m_ref[0], w1_ref[0]) + b1_ref[0])
    h_out[...] = (h_ref[0] + (_dot(u, w2_ref[0]) + b2_ref[0]))[None]


def _expert_spec(shape):
    n = len(shape)
    return pl.BlockSpec((1,) + shape, lambda e: (e,) + (0,) * n)


def _const_spec(shape):
    n = len(shape)
    return pl.BlockSpec(shape, lambda e: (0,) * n)


def _attn_call(a3, h3, wqkv, bqkv, wo, bo):
    return pl.pallas_call(
        _attn_kernel,
        grid=(E,),
        in_specs=[
            _expert_spec((TP, DIM)),
            _expert_spec((TP, DIM)),
            _expert_spec((DIM, 3 * DIM)),
            _expert_spec((1, 3 * DIM)),
            _expert_spec((DIM, DIM)),
            _expert_spec((1, DIM)),
        ],
        out_specs=_expert_spec((TP, DIM)),
        out_shape=jax.ShapeDtypeStruct((E, TP, DIM), jnp.float32),
        compiler_params=pltpu.CompilerParams(
            dimension_semantics=("arbitrary",)),
    )(a3, h3, wqkv, bqkv.reshape(E, 1, 3 * DIM), wo, bo.reshape(E, 1, DIM))


def _mlp_call(m3, h3, w1, b1, w2, b2):
    return pl.pallas_call(
        _mlp_kernel,
        grid=(E,),
        in_specs=[
            _expert_spec((TP, DIM)),
            _expert_spec((TP, DIM)),
            _expert_spec((DIM, MLP)),
            _expert_spec((1, MLP)),
            _expert_spec((MLP, DIM)),
            _expert_spec((1, DIM)),
        ],
        out_specs=_expert_spec((TP, DIM)),
        out_shape=jax.ShapeDtypeStruct((E, TP, DIM), jnp.float32),
        compiler_params=pltpu.CompilerParams(
            dimension_semantics=("arbitrary",)),
    )(m3, h3, w1, b1.reshape(E, 1, MLP), w2, b2.reshape(E, 1, DIM))


@jax.jit
def _run(x, W_rp, b_rp, W_rf, b_rf, W_emb, b_emb, cls_tok, ln1_g, ln1_b,
         Wqkv, bqkv, Wo, bo, ln2_g, ln2_b, W1, b1, W2, b2, lnf_g, lnf_b,
         Wh1, bh1, Wh2, bh2):
    x2 = x.reshape(N, D_IN)

    hemb, g_soft = pl.pallas_call(
        _embed_kernel,
        grid=(E,),
        in_specs=[
            _const_spec((N, D_IN)),
            _const_spec((D_IN, DIM)),
            _const_spec((1, DIM)),
            _const_spec((DIM, E)),
            _const_spec((1, E)),
            _expert_spec((D_IN, DIM)),
            _expert_spec((1, DIM)),
        ],
        out_specs=(_expert_spec((N, DIM)), _const_spec((1, E))),
        out_shape=(jax.ShapeDtypeStruct((E, N, DIM), jnp.float32),
                   jax.ShapeDtypeStruct((1, E), jnp.float32)),
        compiler_params=pltpu.CompilerParams(
            dimension_semantics=("arbitrary",)),
    )(x2, W_rp, b_rp.reshape(1, DIM), W_rf, b_rf.reshape(1, E),
      W_emb, b_emb.reshape(E, 1, DIM))

    h = jnp.concatenate(
        [cls_tok[:, None, :], hemb,
         jnp.zeros((E, TP - N - 1, DIM), jnp.float32)], axis=1)
    h = jax.lax.optimization_barrier(h)

    for l in range(L):
        a = _ln(h, ln1_g[:, l][:, None, :], ln1_b[:, l][:, None, :])
        h = _attn_call(a, h, Wqkv[:, l], bqkv[:, l], Wo[:, l], bo[:, l])
        m = _ln(h, ln2_g[:, l][:, None, :], ln2_b[:, l][:, None, :])
        h = _mlp_call(m, h, W1[:, l], b1[:, l], W2[:, l], b2[:, l])

    hf = _ln(h, lnf_g[:, None, :], lnf_b[:, None, :])
    latents = hf[:, 0][None]
    latent = jnp.sum(latents * g_soft[..., None], axis=1)
    logits = jax.nn.relu(latent @ Wh1 + bh1) @ Wh2 + bh2
    return latent, logits, g_soft


def kernel(x, W_rp, b_rp, W_rf, b_rf, W_emb, b_emb, cls_tok, ln1_g, ln1_b,
           Wqkv, bqkv, Wo, bo, ln2_g, ln2_b, W1, b1, W2, b2, lnf_g, lnf_b,
           Wh1, bh1, Wh2, bh2):
    return _run(x, W_rp, b_rp, W_rf, b_rf, W_emb, b_emb, cls_tok, ln1_g,
                ln1_b, Wqkv, bqkv, Wo, bo, ln2_g, ln2_b, W1, b1, W2, b2,
                lnf_g, lnf_b, Wh1, bh1, Wh2, bh2)
